# Initial kernel scaffold; baseline (speedup 1.0000x reference)
import jax
import jax.numpy as jnp
from jax.experimental import pallas as pl


def kernel(activation):
    raise NotImplementedError("write your pallas kernel here")


# speedup vs baseline: 6.7002x; 6.7002x over previous
import jax
import jax.numpy as jnp
from jax.experimental import pallas as pl
from jax.experimental.pallas import tpu as pltpu

_N, _C, _H, _W = 8, 96, 224, 224


def _pairsum(x, axis, dist):
    ax = axis % x.ndim
    fwd = jnp.roll(x, -dist, axis=ax)
    bwd = jnp.roll(x, dist, axis=ax)
    mshape = tuple(x.shape[i] if i == ax else 1 for i in range(x.ndim))
    idx = jax.lax.broadcasted_iota(jnp.int32, mshape, ax)
    take_fwd = (idx // dist) % 2 == 0
    return x + jnp.where(take_fwd, fwd, bwd)


def _mask22(x):
    s = _pairsum(_pairsum(x, -1, 1), -2, 1)
    return jnp.where(s > 0, x, jnp.zeros_like(x))


def _mask44(x):
    s = _pairsum(_pairsum(x, -1, 1), -1, 2)
    s = _pairsum(_pairsum(s, -2, 1), -2, 2)
    return jnp.where(s > 0, x, jnp.zeros_like(x))


def _mask12(x):
    s = _pairsum(x, -1, 1)
    return jnp.where(s > 0, x, jnp.zeros_like(x))


def _identity(x):
    return x


def _body(x_ref, o_ref):
    c = pl.program_id(0)
    b = ((c >= 32).astype(jnp.int32) + (c >= 64).astype(jnp.int32)
         + (c >= 80).astype(jnp.int32))
    o_ref[...] = jax.lax.switch(b, [_mask22, _mask44, _mask12, _identity],
                                x_ref[...])


def kernel(activation):
    x5 = activation.reshape(_N, _C, _H // 8, 8, _W)
    out = pl.pallas_call(
        _body,
        grid=(_C,),
        in_specs=[pl.BlockSpec((_N, 1, _H // 8, 8, 256),
                               lambda c: (0, c, 0, 0, 0))],
        out_specs=pl.BlockSpec((_N, 1, _H // 8, 8, 256),
                               lambda c: (0, c, 0, 0, 0)),
        out_shape=jax.ShapeDtypeStruct((_N, _C, _H // 8, 8, _W),
                                       activation.dtype),
    )(x5)
    return out.reshape(_N, _C, _H, _W)


# speedup vs baseline: 8.6974x; 1.2981x over previous
import jax
import jax.numpy as jnp
import numpy as np
from jax.experimental import pallas as pl
from jax.experimental.pallas import tpu as pltpu

_N, _C, _H, _W = 8, 96, 224, 224
_WP = 256

_DN = (((1,), (0,)), ((), ()))


def _group_ones(g):
    i = np.arange(_WP)
    m = (i[:, None] // g) == (i[None, :] // g)
    return jnp.asarray(m, dtype=jnp.bfloat16)


def _wsum(x, m_ref):
    shape = x.shape
    x2 = x.reshape(-1, shape[-1])
    lane = jax.lax.broadcasted_iota(jnp.int32, (1, shape[-1]), 1)
    x2 = jnp.where(lane < _W, x2, jnp.zeros_like(x2))
    xh = x2.astype(jnp.bfloat16)
    xl = (x2 - xh.astype(jnp.float32)).astype(jnp.bfloat16)
    m = m_ref[...]
    s = jax.lax.dot_general(xh, m, _DN, preferred_element_type=jnp.float32)
    s = s + jax.lax.dot_general(xl, m, _DN,
                                preferred_element_type=jnp.float32)
    return s.reshape(shape)


def _rowsum(x, dist):
    ax = x.ndim - 2
    fwd = jnp.roll(x, -dist, axis=ax)
    bwd = jnp.roll(x, dist, axis=ax)
    mshape = tuple(x.shape[i] if i == ax else 1 for i in range(x.ndim))
    idx = jax.lax.broadcasted_iota(jnp.int32, mshape, ax)
    take_fwd = (idx // dist) % 2 == 0
    return x + jnp.where(take_fwd, fwd, bwd)


def _body(x_ref, m2_ref, m4_ref, o_ref):
    c = pl.program_id(0)
    b = ((c >= 32).astype(jnp.int32) + (c >= 64).astype(jnp.int32)
         + (c >= 80).astype(jnp.int32))

    def f22(x):
        s = _rowsum(_wsum(x, m2_ref), 1)
        return jnp.where(s > 0, x, jnp.zeros_like(x))

    def f44(x):
        s = _rowsum(_rowsum(_wsum(x, m4_ref), 1), 2)
        return jnp.where(s > 0, x, jnp.zeros_like(x))

    def f12(x):
        s = _wsum(x, m2_ref)
        return jnp.where(s > 0, x, jnp.zeros_like(x))

    def fid(x):
        return x

    o_ref[...] = jax.lax.switch(b, [f22, f44, f12, fid], x_ref[...])


def kernel(activation):
    x5 = activation.reshape(_N, _C, _H // 8, 8, _W)
    out = pl.pallas_call(
        _body,
        grid=(_C,),
        in_specs=[
            pl.BlockSpec((_N, 1, _H // 8, 8, _WP),
                         lambda c: (0, c, 0, 0, 0)),
            pl.BlockSpec((_WP, _WP), lambda c: (0, 0)),
            pl.BlockSpec((_WP, _WP), lambda c: (0, 0)),
        ],
        out_specs=pl.BlockSpec((_N, 1, _H // 8, 8, _WP),
                               lambda c: (0, c, 0, 0, 0)),
        out_shape=jax.ShapeDtypeStruct((_N, _C, _H // 8, 8, _W),
                                       activation.dtype),
    )(x5, _group_ones(2), _group_ones(4))
    return out.reshape(_N, _C, _H, _W)


# speedup vs baseline: 9.4167x; 1.0827x over previous
import jax
import jax.numpy as jnp
import numpy as np
from jax.experimental import pallas as pl
from jax.experimental.pallas import tpu as pltpu

_N, _C, _H, _W = 8, 96, 224, 224
_WP = 224
_CB = 2

_DN = (((1,), (0,)), ((), ()))


def _group_ones(g):
    i = np.arange(_WP)
    m = (i[:, None] // g) == (i[None, :] // g)
    return jnp.asarray(m, dtype=jnp.bfloat16)


def _wsum(x, m_ref):
    shape = x.shape
    x2 = x.reshape(-1, shape[-1])
    xh = x2.astype(jnp.bfloat16)
    xl = (x2 - xh.astype(jnp.float32)).astype(jnp.bfloat16)
    m = m_ref[...]
    s = jax.lax.dot_general(xh, m, _DN, preferred_element_type=jnp.float32)
    s = s + jax.lax.dot_general(xl, m, _DN,
                                preferred_element_type=jnp.float32)
    return s.reshape(shape)


def _rowsum(x, dist):
    ax = x.ndim - 2
    fwd = jnp.roll(x, -dist, axis=ax)
    bwd = jnp.roll(x, dist, axis=ax)
    mshape = tuple(x.shape[i] if i == ax else 1 for i in range(x.ndim))
    idx = jax.lax.broadcasted_iota(jnp.int32, mshape, ax)
    take_fwd = (idx // dist) % 2 == 0
    return x + jnp.where(take_fwd, fwd, bwd)


def _body(x_ref, m2_ref, m4_ref, o_ref):
    c = pl.program_id(0) * _CB
    b = ((c >= 32).astype(jnp.int32) + (c >= 64).astype(jnp.int32)
         + (c >= 80).astype(jnp.int32))

    def f22(x):
        s = _rowsum(_wsum(x, m2_ref), 1)
        return jnp.where(s > 0, x, jnp.zeros_like(x))

    def f44(x):
        s = _rowsum(_rowsum(_wsum(x, m4_ref), 1), 2)
        return jnp.where(s > 0, x, jnp.zeros_like(x))

    def f12(x):
        s = _wsum(x, m2_ref)
        return jnp.where(s > 0, x, jnp.zeros_like(x))

    def fid(x):
        return x

    o_ref[...] = jax.lax.switch(b, [f22, f44, f12, fid], x_ref[...])


def kernel(activation):
    x5 = activation.reshape(_N, _C, _H // 8, 8, _W)
    out = pl.pallas_call(
        _body,
        grid=(_C // _CB,),
        in_specs=[
            pl.BlockSpec((_N, _CB, _H // 8, 8, _WP),
                         lambda c: (0, c, 0, 0, 0)),
            pl.BlockSpec((_WP, _WP), lambda c: (0, 0)),
            pl.BlockSpec((_WP, _WP), lambda c: (0, 0)),
        ],
        out_specs=pl.BlockSpec((_N, _CB, _H // 8, 8, _WP),
                               lambda c: (0, c, 0, 0, 0)),
        out_shape=jax.ShapeDtypeStruct((_N, _C, _H // 8, 8, _W),
                                       activation.dtype),
    )(x5, _group_ones(2), _group_ones(4))
    return out.reshape(_N, _C, _H, _W)


# speedup vs baseline: 9.5470x; 1.0138x over previous
import jax
import jax.numpy as jnp
import numpy as np
from jax.experimental import pallas as pl
from jax.experimental.pallas import tpu as pltpu

_N, _C, _H, _W = 8, 96, 224, 224
_WP = 224
_CB = 4

_DN = (((1,), (0,)), ((), ()))


def _group_ones(g):
    i = np.arange(_WP)
    m = (i[:, None] // g) == (i[None, :] // g)
    return jnp.asarray(m, dtype=jnp.bfloat16)


def _wsum(x, m_ref):
    shape = x.shape
    x2 = x.reshape(-1, shape[-1])
    xh = x2.astype(jnp.bfloat16)
    xl = (x2 - xh.astype(jnp.float32)).astype(jnp.bfloat16)
    m = m_ref[...]
    s = jax.lax.dot_general(xh, m, _DN, preferred_element_type=jnp.float32)
    s = s + jax.lax.dot_general(xl, m, _DN,
                                preferred_element_type=jnp.float32)
    return s.reshape(shape)


def _rowsum(x, dist):
    ax = x.ndim - 2
    fwd = jnp.roll(x, -dist, axis=ax)
    bwd = jnp.roll(x, dist, axis=ax)
    mshape = tuple(x.shape[i] if i == ax else 1 for i in range(x.ndim))
    idx = jax.lax.broadcasted_iota(jnp.int32, mshape, ax)
    take_fwd = (idx // dist) % 2 == 0
    return x + jnp.where(take_fwd, fwd, bwd)


def _body(x_ref, m2_ref, m4_ref, o_ref):
    c = pl.program_id(0) * _CB
    b = ((c >= 32).astype(jnp.int32) + (c >= 64).astype(jnp.int32)
         + (c >= 80).astype(jnp.int32))

    def f22(x):
        s = _rowsum(_wsum(x, m2_ref), 1)
        return jnp.where(s > 0, x, jnp.zeros_like(x))

    def f44(x):
        s = _rowsum(_rowsum(_wsum(x, m4_ref), 1), 2)
        return jnp.where(s > 0, x, jnp.zeros_like(x))

    def f12(x):
        s = _wsum(x, m2_ref)
        return jnp.where(s > 0, x, jnp.zeros_like(x))

    def fid(x):
        return x

    o_ref[...] = jax.lax.switch(b, [f22, f44, f12, fid], x_ref[...])


def kernel(activation):
    x5 = activation.reshape(_N, _C, _H // 8, 8, _W)
    out = pl.pallas_call(
        _body,
        grid=(_C // _CB,),
        in_specs=[
            pl.BlockSpec((_N, _CB, _H // 8, 8, _WP),
                         lambda c: (0, c, 0, 0, 0)),
            pl.BlockSpec((_WP, _WP), lambda c: (0, 0)),
            pl.BlockSpec((_WP, _WP), lambda c: (0, 0)),
        ],
        out_specs=pl.BlockSpec((_N, _CB, _H // 8, 8, _WP),
                               lambda c: (0, c, 0, 0, 0)),
        out_shape=jax.ShapeDtypeStruct((_N, _C, _H // 8, 8, _W),
                                       activation.dtype),
    )(x5, _group_ones(2), _group_ones(4))
    return out.reshape(_N, _C, _H, _W)
